# baseline (speedup 1.0000x reference)
import functools

import jax
import jax.numpy as jnp
from jax.experimental import pallas as pl
from jax.experimental.pallas import tpu as pltpu

_K = 256
_BLOCK_ROWS = 32


def _topk_mask_kernel(x_ref, o_ref, *, k):
    x = x_ref[...]
    rows, cols = x.shape
    kI = jnp.int32(k)
    log_k = jnp.log(jnp.float32(k))

    i32 = jax.lax.bitcast_convert_type(x, jnp.int32)
    key = i32 ^ (jax.lax.shift_right_arithmetic(i32, 31) & jnp.int32(0x7FFFFFFF))
    khi = jax.lax.shift_right_arithmetic(key, 16).astype(jnp.int16)
    klos = (key ^ jnp.int32(0x8000)).astype(jnp.int16)

    ch = min(2048, cols)
    one16 = jnp.int16(1)
    zero16 = jnp.int16(0)

    def chunksum(ind):
        acc = ind[:, :ch]
        for j in range(1, cols // ch):
            acc = acc + ind[:, j * ch:(j + 1) * ch]
        return jnp.sum(acc.astype(jnp.int32), axis=1, keepdims=True)

    def shape1(v):
        return jnp.full((rows, 1), v, jnp.int32)

    def search(count_fn, init, max_it, first_zero, log_interp):
        def step(carry):
            it, lo, hi, c_lo, c_hi = carry
            w = hi - lo
            cand_mid = jnp.maximum(
                lo + jax.lax.shift_right_logical(w, jnp.int32(1)), lo + 1)
            if log_interp:
                cl = jnp.log(c_lo.astype(jnp.float32))
                frac = (cl - log_k) / (cl - jnp.log(
                    c_hi.astype(jnp.float32) + 0.5) + 1e-9)
            else:
                frac = ((c_lo - kI).astype(jnp.float32)
                        / (c_lo - c_hi).astype(jnp.float32))
            cand_f = lo.astype(jnp.float32) + frac * w.astype(jnp.float32)
            margin = jax.lax.shift_right_logical(w, jnp.int32(4))
            cand_int = jnp.clip(cand_f.astype(jnp.int32),
                                lo + jnp.maximum(margin, 1), hi - margin)
            use_mid = jnp.logical_or(it == 1, it % 6 == 5)
            cand = jnp.where(use_mid, cand_mid, cand_int)
            if first_zero:
                cand = jnp.where(it == 0, shape1(0), cand)
            c = count_fn(cand)
            active = hi > lo
            ge = jnp.logical_and(active, c >= kI)
            lt = jnp.logical_and(active, c < kI)
            hit = jnp.logical_and(active, c == kI)
            lo = jnp.where(ge, cand, lo)
            c_lo = jnp.where(ge, c, c_lo)
            hi = jnp.where(hit, cand, jnp.where(lt, cand - 1, hi))
            c_hi = jnp.where(lt, c, c_hi)
            return it + 1, lo, hi, c_lo, c_hi

        body = step

        def cond(carry):
            it, lo, hi, _, _ = carry
            return jnp.logical_and(it < max_it, jnp.any(hi > lo))

        return jax.lax.while_loop(cond, body, init)

    def count_hi(cand):
        c16 = cand.astype(jnp.int16)
        return chunksum(jnp.where(khi >= c16, one16, zero16))

    initA = (jnp.int32(0), shape1(-32768), shape1(32767), shape1(cols),
             shape1(0))
    _, h, _, c_loA, c_hiA = search(count_hi, initA, 24, True, True)

    frozen = c_loA == kI
    h16 = h.astype(jnp.int16)
    eq16 = jnp.where(khi == h16, one16, zero16)

    def count_lo(cand):
        c16 = cand.astype(jnp.int16)
        return c_hiA + chunksum(jnp.where(klos >= c16, eq16, zero16))

    initB = (jnp.int32(0), shape1(-32768),
             jnp.where(frozen, shape1(-32768), shape1(32767)), c_loA, c_hiA)
    _, lo_b, _, _, _ = search(count_lo, initB, 24, False, False)

    lbits = (lo_b ^ jnp.int32(0x8000)) & jnp.int32(0xFFFF)
    hshift = jax.lax.shift_left(h, jnp.int32(16))
    thresh_key = jnp.where(frozen, hshift, hshift | lbits)
    tf = jax.lax.bitcast_convert_type(
        jnp.maximum(thresh_key, jnp.int32(1)), jnp.float32)
    o_ref[...] = jnp.where(x >= tf, x, jnp.float32(0.0))


def kernel(x):
    rows, cols = x.shape
    br = min(_BLOCK_ROWS, rows)
    return pl.pallas_call(
        functools.partial(_topk_mask_kernel, k=_K),
        grid=(rows // br,),
        in_specs=[pl.BlockSpec((br, cols), lambda i: (i, 0))],
        out_specs=pl.BlockSpec((br, cols), lambda i: (i, 0)),
        out_shape=jax.ShapeDtypeStruct(x.shape, x.dtype),
        compiler_params=pltpu.CompilerParams(
            dimension_semantics=("parallel",)),
    )(x)


# speedup vs baseline: 1.0730x; 1.0730x over previous
import functools

import jax
import jax.numpy as jnp
from jax.experimental import pallas as pl
from jax.experimental.pallas import tpu as pltpu

_K = 256
_BLOCK_ROWS = 64


def _topk_mask_kernel(x_ref, o_ref, *, k):
    x = x_ref[...]
    rows, cols = x.shape
    kI = jnp.int32(k)
    log_k = jnp.log(jnp.float32(k))

    i32 = jax.lax.bitcast_convert_type(x, jnp.int32)
    key = i32 ^ (jax.lax.shift_right_arithmetic(i32, 31) & jnp.int32(0x7FFFFFFF))
    khi = jax.lax.shift_right_arithmetic(key, 16).astype(jnp.int16)
    klos = (key ^ jnp.int32(0x8000)).astype(jnp.int16)

    ch = min(2048, cols)
    one16 = jnp.int16(1)
    zero16 = jnp.int16(0)

    def chunksum(ind):
        acc = ind[:, :ch]
        for j in range(1, cols // ch):
            acc = acc + ind[:, j * ch:(j + 1) * ch]
        return jnp.sum(acc.astype(jnp.int32), axis=1, keepdims=True)

    def shape1(v):
        return jnp.full((rows, 1), v, jnp.int32)

    def search(count_fn, init, max_it, first_zero, log_interp):
        def step(carry):
            it, lo, hi, c_lo, c_hi = carry
            w = hi - lo
            cand_mid = jnp.maximum(
                lo + jax.lax.shift_right_logical(w, jnp.int32(1)), lo + 1)
            if log_interp:
                cl = jnp.log(c_lo.astype(jnp.float32))
                frac = (cl - log_k) / (cl - jnp.log(
                    c_hi.astype(jnp.float32) + 0.5) + 1e-9)
            else:
                frac = ((c_lo - kI).astype(jnp.float32)
                        / (c_lo - c_hi).astype(jnp.float32))
            cand_f = lo.astype(jnp.float32) + frac * w.astype(jnp.float32)
            margin = jax.lax.shift_right_logical(w, jnp.int32(4))
            cand_int = jnp.clip(cand_f.astype(jnp.int32),
                                lo + jnp.maximum(margin, 1), hi - margin)
            use_mid = jnp.logical_or(it == 1, it % 6 == 5)
            cand = jnp.where(use_mid, cand_mid, cand_int)
            if first_zero:
                cand = jnp.where(it == 0, shape1(0), cand)
            c = count_fn(cand)
            active = hi > lo
            ge = jnp.logical_and(active, c >= kI)
            lt = jnp.logical_and(active, c < kI)
            hit = jnp.logical_and(active, c == kI)
            lo = jnp.where(ge, cand, lo)
            c_lo = jnp.where(ge, c, c_lo)
            hi = jnp.where(hit, cand, jnp.where(lt, cand - 1, hi))
            c_hi = jnp.where(lt, c, c_hi)
            return it + 1, lo, hi, c_lo, c_hi

        body = step

        def cond(carry):
            it, lo, hi, _, _ = carry
            return jnp.logical_and(it < max_it, jnp.any(hi > lo))

        return jax.lax.while_loop(cond, body, init)

    def count_hi(cand):
        c16 = cand.astype(jnp.int16)
        return chunksum(jnp.where(khi >= c16, one16, zero16))

    initA = (jnp.int32(0), shape1(-32768), shape1(32767), shape1(cols),
             shape1(0))
    _, h, _, c_loA, c_hiA = search(count_hi, initA, 24, True, True)

    frozen = c_loA == kI
    h16 = h.astype(jnp.int16)
    eq16 = jnp.where(khi == h16, one16, zero16)

    def count_lo(cand):
        c16 = cand.astype(jnp.int16)
        return c_hiA + chunksum(jnp.where(klos >= c16, eq16, zero16))

    initB = (jnp.int32(0), shape1(-32768),
             jnp.where(frozen, shape1(-32768), shape1(32767)), c_loA, c_hiA)
    _, lo_b, _, _, _ = search(count_lo, initB, 24, False, False)

    lbits = (lo_b ^ jnp.int32(0x8000)) & jnp.int32(0xFFFF)
    hshift = jax.lax.shift_left(h, jnp.int32(16))
    thresh_key = jnp.where(frozen, hshift, hshift | lbits)
    tf = jax.lax.bitcast_convert_type(
        jnp.maximum(thresh_key, jnp.int32(1)), jnp.float32)
    o_ref[...] = jnp.where(x >= tf, x, jnp.float32(0.0))


def kernel(x):
    rows, cols = x.shape
    br = min(_BLOCK_ROWS, rows)
    return pl.pallas_call(
        functools.partial(_topk_mask_kernel, k=_K),
        grid=(rows // br,),
        in_specs=[pl.BlockSpec((br, cols), lambda i: (i, 0))],
        out_specs=pl.BlockSpec((br, cols), lambda i: (i, 0)),
        out_shape=jax.ShapeDtypeStruct(x.shape, x.dtype),
        compiler_params=pltpu.CompilerParams(
            dimension_semantics=("parallel",)),
    )(x)
